# baseline (speedup 1.0000x reference)
import jax
import jax.numpy as jnp
from jax import lax
from jax.experimental import pallas as pl
from jax.experimental.pallas import tpu as pltpu

B, S, D, E, TOPK = 4, 8192, 768, 64, 2
EPS = 1e-9
S_BLK = 2048


def _router_block(x_ref, w_ref, mask_ref, idx_ref, rprobs_ref, probs_ref):
    x = x_ref[0]
    w = w_ref[...]
    logits = lax.dot_general(w, x, (((1,), (1,)), ((), ())),
                             preferred_element_type=jnp.float32)
    m = jnp.max(logits, axis=0, keepdims=True)
    ex = jnp.exp(logits - m)
    probs = ex / jnp.sum(ex, axis=0, keepdims=True)
    probs = jnp.clip(probs + EPS, EPS, 1.0 - EPS)

    eidx = lax.broadcasted_iota(
        jnp.int32, probs.shape, 0).astype(jnp.float32)
    big = jnp.float32(E)
    m1 = jnp.max(probs, axis=0, keepdims=True)
    i1 = jnp.min(jnp.where(probs == m1, eidx, big), axis=0, keepdims=True)
    masked = jnp.where(eidx == i1, -1.0, probs)
    m2 = jnp.max(masked, axis=0, keepdims=True)
    i2 = jnp.min(jnp.where(masked == m2, eidx, big), axis=0, keepdims=True)

    is1 = eidx == i1
    is2 = eidx == i2
    mask_ref[0] = (is1 | is2).astype(jnp.float32)
    num = jnp.where(is1, m1, 0.0) + jnp.where(is2, m2, 0.0)
    rprobs_ref[0] = num / (m1 + m2)
    probs_ref[0] = probs
    idx_ref[0] = jnp.concatenate([i1, i2], axis=0).astype(jnp.int32)


def kernel(inputs, cond, W):
    del cond
    grid = (B, S // S_BLK)
    mask_t, idx_t, rprobs_t, probs_t = pl.pallas_call(
        _router_block,
        grid=grid,
        in_specs=[
            pl.BlockSpec((1, S_BLK, D), lambda b, s: (b, s, 0)),
            pl.BlockSpec((E, D), lambda b, s: (0, 0)),
        ],
        out_specs=[
            pl.BlockSpec((1, E, S_BLK), lambda b, s: (b, 0, s)),
            pl.BlockSpec((1, TOPK, S_BLK), lambda b, s: (b, 0, s)),
            pl.BlockSpec((1, E, S_BLK), lambda b, s: (b, 0, s)),
            pl.BlockSpec((1, E, S_BLK), lambda b, s: (b, 0, s)),
        ],
        out_shape=[
            jax.ShapeDtypeStruct((B, E, S), jnp.float32),
            jax.ShapeDtypeStruct((B, TOPK, S), jnp.int32),
            jax.ShapeDtypeStruct((B, E, S), jnp.float32),
            jax.ShapeDtypeStruct((B, E, S), jnp.float32),
        ],
        compiler_params=pltpu.CompilerParams(
            dimension_semantics=("parallel", "parallel"),
        ),
    )(inputs, W)
    tr = lambda a: jnp.transpose(a, (0, 2, 1))
    return tr(mask_t), tr(idx_t), tr(rprobs_t), tr(probs_t)


# speedup vs baseline: 1.0490x; 1.0490x over previous
import jax
import jax.numpy as jnp
from jax import lax
from jax.experimental import pallas as pl
from jax.experimental.pallas import tpu as pltpu

B, S, D, E, TOPK = 4, 8192, 768, 64, 2
EPS = 1e-9
S_BLK = 4096


def _router_block(x_ref, w_ref, mask_ref, idx_ref, rprobs_ref, probs_ref):
    x = x_ref[0]
    w = w_ref[...]
    logits = lax.dot_general(w, x, (((1,), (1,)), ((), ())),
                             preferred_element_type=jnp.float32)
    m = jnp.max(logits, axis=0, keepdims=True)
    ex = jnp.exp(logits - m)
    probs = ex / jnp.sum(ex, axis=0, keepdims=True)
    probs = jnp.clip(probs + EPS, EPS, 1.0 - EPS)

    eidx = lax.broadcasted_iota(
        jnp.int32, probs.shape, 0).astype(jnp.float32)
    big = jnp.float32(E)
    m1 = jnp.max(probs, axis=0, keepdims=True)
    i1 = jnp.min(jnp.where(probs == m1, eidx, big), axis=0, keepdims=True)
    masked = jnp.where(eidx == i1, -1.0, probs)
    m2 = jnp.max(masked, axis=0, keepdims=True)
    i2 = jnp.min(jnp.where(masked == m2, eidx, big), axis=0, keepdims=True)

    is1 = eidx == i1
    is2 = eidx == i2
    mask_ref[0] = (is1 | is2).astype(jnp.float32)
    num = jnp.where(is1, m1, 0.0) + jnp.where(is2, m2, 0.0)
    rprobs_ref[0] = num / (m1 + m2)
    probs_ref[0] = probs
    idx_ref[0] = jnp.concatenate([i1, i2], axis=0).astype(jnp.int32)


def kernel(inputs, cond, W):
    del cond
    grid = (B, S // S_BLK)
    mask_t, idx_t, rprobs_t, probs_t = pl.pallas_call(
        _router_block,
        grid=grid,
        in_specs=[
            pl.BlockSpec((1, S_BLK, D), lambda b, s: (b, s, 0)),
            pl.BlockSpec((E, D), lambda b, s: (0, 0)),
        ],
        out_specs=[
            pl.BlockSpec((1, E, S_BLK), lambda b, s: (b, 0, s)),
            pl.BlockSpec((1, TOPK, S_BLK), lambda b, s: (b, 0, s)),
            pl.BlockSpec((1, E, S_BLK), lambda b, s: (b, 0, s)),
            pl.BlockSpec((1, E, S_BLK), lambda b, s: (b, 0, s)),
        ],
        out_shape=[
            jax.ShapeDtypeStruct((B, E, S), jnp.float32),
            jax.ShapeDtypeStruct((B, TOPK, S), jnp.int32),
            jax.ShapeDtypeStruct((B, E, S), jnp.float32),
            jax.ShapeDtypeStruct((B, E, S), jnp.float32),
        ],
        compiler_params=pltpu.CompilerParams(
            dimension_semantics=("parallel", "parallel"),
        ),
    )(inputs, W)
    tr = lambda a: jnp.transpose(a, (0, 2, 1))
    return tr(mask_t), tr(idx_t), tr(rprobs_t), tr(probs_t)


# speedup vs baseline: 1.0500x; 1.0009x over previous
import jax
import jax.numpy as jnp
from jax import lax
from jax.experimental import pallas as pl
from jax.experimental.pallas import tpu as pltpu

B, S, D, E, TOPK = 4, 8192, 768, 64, 2
EPS = 1e-9
S_BLK = 4096


def _router_block(x_ref, w_ref, mask_ref, idx_ref, rprobs_ref, probs_ref):
    x = x_ref[0]
    w = w_ref[...]
    logits = lax.dot_general(w, x, (((1,), (1,)), ((), ())),
                             preferred_element_type=jnp.float32)
    m = jnp.max(logits, axis=0, keepdims=True)
    ex = jnp.exp(logits - m)
    probs = ex / jnp.sum(ex, axis=0, keepdims=True)
    probs = jnp.clip(probs + EPS, EPS, 1.0 - EPS)

    eidx = lax.broadcasted_iota(
        jnp.int32, probs.shape, 0).astype(jnp.float32)
    big = jnp.float32(E)
    m1 = jnp.max(probs, axis=0, keepdims=True)
    i1 = jnp.min(jnp.where(probs == m1, eidx, big), axis=0, keepdims=True)
    masked = jnp.where(eidx == i1, -1.0, probs)
    m2 = jnp.max(masked, axis=0, keepdims=True)
    i2 = jnp.min(jnp.where(masked == m2, eidx, big), axis=0, keepdims=True)

    is1 = eidx == i1
    is2 = eidx == i2
    mask_ref[0] = (is1 | is2).astype(jnp.float32)
    num = jnp.where(is1, m1, 0.0) + jnp.where(is2, m2, 0.0)
    rprobs_ref[0] = num / (m1 + m2)
    probs_ref[0] = probs
    idx_ref[0] = jnp.concatenate([i1, i2], axis=0).astype(jnp.int32)


def kernel(inputs, cond, W):
    del cond
    grid = (B, S // S_BLK)
    mask_t, idx_t, rprobs_t, probs_t = pl.pallas_call(
        _router_block,
        grid=grid,
        in_specs=[
            pl.BlockSpec((1, S_BLK, D), lambda b, s: (b, s, 0)),
            pl.BlockSpec((E, D), lambda b, s: (0, 0)),
        ],
        out_specs=[
            pl.BlockSpec((1, E, S_BLK), lambda b, s: (b, 0, s)),
            pl.BlockSpec((1, TOPK, S_BLK), lambda b, s: (b, 0, s)),
            pl.BlockSpec((1, E, S_BLK), lambda b, s: (b, 0, s)),
            pl.BlockSpec((1, E, S_BLK), lambda b, s: (b, 0, s)),
        ],
        out_shape=[
            jax.ShapeDtypeStruct((B, E, S), jnp.float32),
            jax.ShapeDtypeStruct((B, TOPK, S), jnp.int32),
            jax.ShapeDtypeStruct((B, E, S), jnp.float32),
            jax.ShapeDtypeStruct((B, E, S), jnp.float32),
        ],
        compiler_params=pltpu.CompilerParams(
            dimension_semantics=("parallel", "arbitrary"),
        ),
    )(inputs, W)
    tr = lambda a: jnp.transpose(a, (0, 2, 1))
    return tr(mask_t), tr(idx_t), tr(rprobs_t), tr(probs_t)
